# Initial kernel scaffold; baseline (speedup 1.0000x reference)
import functools

import jax
import jax.numpy as jnp
from jax import lax
from jax.experimental import pallas as pl
from jax.experimental.pallas import tpu as pltpu
from jax.experimental.pallas import tpu_sc as plsc

NC = 2
NS = 16
NW = NC * NS
D = 128
E_TOTAL = 320000
EPW = E_TOTAL // NW
CHUNK = 80
NCHUNK = EPW // CHUNK


def _dot_body(ei_hbm, ej_hbm, z_hbm, out_hbm,
              idxi_v, idxj_v, rowsi_v, rowsj_v, outv, semi, semj):
    wid = lax.axis_index("s") * NC + lax.axis_index("c")

    def chunk_body(c, carry):
        base = wid * EPW + c * CHUNK
        pltpu.sync_copy(ei_hbm.at[pl.ds(base, CHUNK)], idxi_v)
        pltpu.sync_copy(ej_hbm.at[pl.ds(base, CHUNK)], idxj_v)
        cpi = pltpu.async_copy(z_hbm.at[idxi_v], rowsi_v, semi)
        cpj = pltpu.async_copy(z_hbm.at[idxj_v], rowsj_v, semj)
        cpi.wait()
        cpj.wait()

        def edge_body(e, carry2):
            acc = rowsi_v[e, pl.ds(0, 16)] * rowsj_v[e, pl.ds(0, 16)]
            for kk in range(1, 8):
                acc = acc + (rowsi_v[e, pl.ds(kk * 16, 16)]
                             * rowsj_v[e, pl.ds(kk * 16, 16)])
            outv[e] = jnp.sum(acc)
            return carry2

        lax.fori_loop(0, CHUNK, edge_body, 0, unroll=2)
        pltpu.sync_copy(outv, out_hbm.at[pl.ds(base, CHUNK)])
        return carry

    lax.fori_loop(0, NCHUNK, chunk_body, 0)


@jax.jit
def kernel(z, edge_index):
    ei = edge_index[0].astype(jnp.int32)
    ej = edge_index[1].astype(jnp.int32)
    mesh = plsc.VectorSubcoreMesh(core_axis_name="c", subcore_axis_name="s")
    f = functools.partial(
        pl.kernel,
        mesh=mesh,
        out_type=jax.ShapeDtypeStruct((E_TOTAL,), jnp.float32),
        scratch_types=[
            pltpu.VMEM((CHUNK,), jnp.int32),
            pltpu.VMEM((CHUNK,), jnp.int32),
            pltpu.VMEM((CHUNK, D), jnp.float32),
            pltpu.VMEM((CHUNK, D), jnp.float32),
            pltpu.VMEM((CHUNK,), jnp.float32),
            pltpu.SemaphoreType.DMA,
            pltpu.SemaphoreType.DMA,
        ],
    )(_dot_body)
    return f(ei, ej, z)


# speedup vs baseline: 2.6105x; 2.6105x over previous
import functools

import jax
import jax.numpy as jnp
from jax import lax
from jax.experimental import pallas as pl
from jax.experimental.pallas import tpu as pltpu
from jax.experimental.pallas import tpu_sc as plsc

NC = 2
NS = 16
NW = NC * NS
D = 128
E_TOTAL = 320000
EPW = E_TOTAL // NW
CHUNK = 80
NCHUNK = EPW // CHUNK


def _dot_body(ei_hbm, ej_hbm, z_hbm, out_hbm,
              idxi_v, idxj_v, rowsi_v, rowsj_v, outv, mat_v, semi, semj):
    wid = lax.axis_index("s") * NC + lax.axis_index("c")

    def chunk_body(c, carry):
        base = wid * EPW + c * CHUNK
        pltpu.sync_copy(ei_hbm.at[pl.ds(base, CHUNK)], idxi_v)
        pltpu.sync_copy(ej_hbm.at[pl.ds(base, CHUNK)], idxj_v)
        cpi = pltpu.async_copy(z_hbm.at[idxi_v], rowsi_v, semi)
        cpj = pltpu.async_copy(z_hbm.at[idxj_v], rowsj_v, semj)
        cpi.wait()
        cpj.wait()

        lane = lax.iota(jnp.int32, 16)

        def group_body(g, carry2):
            tot = jnp.zeros((16,), jnp.float32)
            for ee in range(16):
                e = g * 16 + ee
                acc = rowsi_v[e, pl.ds(0, 16)] * rowsj_v[e, pl.ds(0, 16)]
                for kk in range(1, 8):
                    acc = acc + (rowsi_v[e, pl.ds(kk * 16, 16)]
                                 * rowsj_v[e, pl.ds(kk * 16, 16)])
                tot = jnp.where(lane == ee, jnp.sum(acc), tot)
            outv[pl.ds(g * 16, 16)] = tot
            return carry2

        lax.fori_loop(0, CHUNK // 16, group_body, 0)
        pltpu.sync_copy(outv, out_hbm.at[pl.ds(base, CHUNK)])
        return carry

    lax.fori_loop(0, NCHUNK, chunk_body, 0)


@jax.jit
def kernel(z, edge_index):
    ei = edge_index[0].astype(jnp.int32)
    ej = edge_index[1].astype(jnp.int32)
    mesh = plsc.VectorSubcoreMesh(core_axis_name="c", subcore_axis_name="s")
    f = functools.partial(
        pl.kernel,
        mesh=mesh,
        out_type=jax.ShapeDtypeStruct((E_TOTAL,), jnp.float32),
        scratch_types=[
            pltpu.VMEM((CHUNK,), jnp.int32),
            pltpu.VMEM((CHUNK,), jnp.int32),
            pltpu.VMEM((CHUNK, D), jnp.float32),
            pltpu.VMEM((CHUNK, D), jnp.float32),
            pltpu.VMEM((CHUNK,), jnp.float32),
            pltpu.VMEM((16 * 17, ), jnp.float32),
            pltpu.SemaphoreType.DMA,
            pltpu.SemaphoreType.DMA,
        ],
        compiler_params=pltpu.CompilerParams(needs_layout_passes=False),
    )(_dot_body)
    return f(ei, ej, z)


# speedup vs baseline: 4.1200x; 1.5782x over previous
import functools

import jax
import jax.numpy as jnp
from jax import lax
from jax.experimental import pallas as pl
from jax.experimental.pallas import tpu as pltpu
from jax.experimental.pallas import tpu_sc as plsc

NC = 2
NS = 16
NW = NC * NS
D = 128
E_TOTAL = 320000
EPW = E_TOTAL // NW
CHUNK = 80
NCHUNK = EPW // CHUNK


def _dot_body(ei_hbm, ej_hbm, z_hbm, out_hbm,
              idxi_all, idxj_all, ri_a, rj_a, ri_b, rj_b, outv,
              si_a, sj_a, si_b, sj_b):
    wid = lax.axis_index("s") * NC + lax.axis_index("c")
    ebase = wid * EPW
    pltpu.sync_copy(ei_hbm.at[pl.ds(ebase, EPW)], idxi_all)
    pltpu.sync_copy(ej_hbm.at[pl.ds(ebase, EPW)], idxj_all)
    lane = lax.iota(jnp.int32, 16)

    def start(c, ri, rj, si, sj):
        pltpu.async_copy(z_hbm.at[idxi_all.at[pl.ds(c * CHUNK, CHUNK)]], ri, si)
        pltpu.async_copy(z_hbm.at[idxj_all.at[pl.ds(c * CHUNK, CHUNK)]], rj, sj)

    def wait(c, ri, rj, si, sj):
        pltpu.make_async_copy(
            z_hbm.at[idxi_all.at[pl.ds(c * CHUNK, CHUNK)]], ri, si).wait()
        pltpu.make_async_copy(
            z_hbm.at[idxj_all.at[pl.ds(c * CHUNK, CHUNK)]], rj, sj).wait()

    def compute(c, ri, rj):
        def group_body(g, carry):
            tot = jnp.zeros((16,), jnp.float32)
            for ee in range(16):
                e = g * 16 + ee
                acc = ri[e, pl.ds(0, 16)] * rj[e, pl.ds(0, 16)]
                for kk in range(1, 8):
                    acc = acc + (ri[e, pl.ds(kk * 16, 16)]
                                 * rj[e, pl.ds(kk * 16, 16)])
                tot = jnp.where(lane == ee, jnp.sum(acc), tot)
            outv[pl.ds(c * CHUNK + g * 16, 16)] = tot
            return carry

        lax.fori_loop(0, CHUNK // 16, group_body, 0)

    start(0, ri_a, rj_a, si_a, sj_a)

    def body2(t2, carry):
        t = 2 * t2
        start(t + 1, ri_b, rj_b, si_b, sj_b)
        wait(t, ri_a, rj_a, si_a, sj_a)
        compute(t, ri_a, rj_a)
        start(t + 2, ri_a, rj_a, si_a, sj_a)
        wait(t + 1, ri_b, rj_b, si_b, sj_b)
        compute(t + 1, ri_b, rj_b)
        return carry

    lax.fori_loop(0, (NCHUNK - 1) // 2, body2, 0)
    wait(NCHUNK - 1, ri_a, rj_a, si_a, sj_a)
    compute(NCHUNK - 1, ri_a, rj_a)
    pltpu.sync_copy(outv, out_hbm.at[pl.ds(ebase, EPW)])


@jax.jit
def kernel(z, edge_index):
    ei = edge_index[0].astype(jnp.int32)
    ej = edge_index[1].astype(jnp.int32)
    mesh = plsc.VectorSubcoreMesh(core_axis_name="c", subcore_axis_name="s")
    f = functools.partial(
        pl.kernel,
        mesh=mesh,
        out_type=jax.ShapeDtypeStruct((E_TOTAL,), jnp.float32),
        scratch_types=[
            pltpu.VMEM((EPW,), jnp.int32),
            pltpu.VMEM((EPW,), jnp.int32),
            pltpu.VMEM((CHUNK, D), jnp.float32),
            pltpu.VMEM((CHUNK, D), jnp.float32),
            pltpu.VMEM((CHUNK, D), jnp.float32),
            pltpu.VMEM((CHUNK, D), jnp.float32),
            pltpu.VMEM((EPW,), jnp.float32),
            pltpu.SemaphoreType.DMA,
            pltpu.SemaphoreType.DMA,
            pltpu.SemaphoreType.DMA,
            pltpu.SemaphoreType.DMA,
        ],
        compiler_params=pltpu.CompilerParams(needs_layout_passes=False),
    )(_dot_body)
    return f(ei, ej, z)


# speedup vs baseline: 4.3487x; 1.0555x over previous
import functools

import jax
import jax.numpy as jnp
from jax import lax
from jax.experimental import pallas as pl
from jax.experimental.pallas import tpu as pltpu
from jax.experimental.pallas import tpu_sc as plsc

NC = 2
NS = 16
NW = NC * NS
D = 128
E_TOTAL = 320000
EPW = E_TOTAL // NW
CHUNK = 80
NCHUNK = EPW // CHUNK


def _dot_body(ei_hbm, ej_hbm, z_hbm, out_hbm,
              idxi_all, idxj_all, ri_a, rj_a, ri_b, rj_b, outv,
              si_a, sj_a, si_b, sj_b):
    wid = lax.axis_index("s") * NC + lax.axis_index("c")
    ebase = wid * EPW
    pltpu.sync_copy(ei_hbm.at[pl.ds(ebase, EPW)], idxi_all)
    pltpu.sync_copy(ej_hbm.at[pl.ds(ebase, EPW)], idxj_all)
    lane = lax.iota(jnp.int32, 16)

    def start(c, ri, rj, si, sj):
        pltpu.async_copy(z_hbm.at[idxi_all.at[pl.ds(c * CHUNK, CHUNK)]], ri, si)
        pltpu.async_copy(z_hbm.at[idxj_all.at[pl.ds(c * CHUNK, CHUNK)]], rj, sj)

    def wait(c, ri, rj, si, sj):
        pltpu.make_async_copy(
            z_hbm.at[idxi_all.at[pl.ds(c * CHUNK, CHUNK)]], ri, si).wait()
        pltpu.make_async_copy(
            z_hbm.at[idxj_all.at[pl.ds(c * CHUNK, CHUNK)]], rj, sj).wait()

    def compute(c, ri, rj):
        def group_body(g, carry):
            e_idx = lane + g * 16
            acc = jnp.zeros((16,), jnp.float32)
            for blk in range(D // 16):
                for t in range(16):
                    dv = (lane ^ t) + blk * 16
                    acc = acc + plsc.load_gather(ri, [e_idx, dv]) \
                        * plsc.load_gather(rj, [e_idx, dv])
            outv[pl.ds(c * CHUNK + g * 16, 16)] = acc
            return carry

        lax.fori_loop(0, CHUNK // 16, group_body, 0)

    start(0, ri_a, rj_a, si_a, sj_a)

    def body2(t2, carry):
        t = 2 * t2
        start(t + 1, ri_b, rj_b, si_b, sj_b)
        wait(t, ri_a, rj_a, si_a, sj_a)
        compute(t, ri_a, rj_a)
        start(t + 2, ri_a, rj_a, si_a, sj_a)
        wait(t + 1, ri_b, rj_b, si_b, sj_b)
        compute(t + 1, ri_b, rj_b)
        return carry

    lax.fori_loop(0, (NCHUNK - 1) // 2, body2, 0)
    wait(NCHUNK - 1, ri_a, rj_a, si_a, sj_a)
    compute(NCHUNK - 1, ri_a, rj_a)
    pltpu.sync_copy(outv, out_hbm.at[pl.ds(ebase, EPW)])


@jax.jit
def kernel(z, edge_index):
    ei = edge_index[0].astype(jnp.int32)
    ej = edge_index[1].astype(jnp.int32)
    mesh = plsc.VectorSubcoreMesh(core_axis_name="c", subcore_axis_name="s")
    f = functools.partial(
        pl.kernel,
        mesh=mesh,
        out_type=jax.ShapeDtypeStruct((E_TOTAL,), jnp.float32),
        scratch_types=[
            pltpu.VMEM((EPW,), jnp.int32),
            pltpu.VMEM((EPW,), jnp.int32),
            pltpu.VMEM((CHUNK, D), jnp.float32),
            pltpu.VMEM((CHUNK, D), jnp.float32),
            pltpu.VMEM((CHUNK, D), jnp.float32),
            pltpu.VMEM((CHUNK, D), jnp.float32),
            pltpu.VMEM((EPW,), jnp.float32),
            pltpu.SemaphoreType.DMA,
            pltpu.SemaphoreType.DMA,
            pltpu.SemaphoreType.DMA,
            pltpu.SemaphoreType.DMA,
        ],
        compiler_params=pltpu.CompilerParams(needs_layout_passes=False),
    )(_dot_body)
    return f(ei, ej, z)


# speedup vs baseline: 9.4703x; 2.1777x over previous
import functools

import jax
import jax.numpy as jnp
from jax import lax
from jax.experimental import pallas as pl
from jax.experimental.pallas import tpu as pltpu
from jax.experimental.pallas import tpu_sc as plsc

NC = 2
NS = 16
NW = NC * NS
D = 128
DP = D // 2
E_TOTAL = 320000
EPW = E_TOTAL // NW
CHUNK = 80
NCHUNK = EPW // CHUNK


def _dot_body(ei_hbm, ej_hbm, z_hbm, out_hbm,
              idxi_all, idxj_all, ri_a, rj_a, ri_b, rj_b, outv,
              si_a, sj_a, si_b, sj_b):
    wid = lax.axis_index("s") * NC + lax.axis_index("c")
    ebase = wid * EPW
    pltpu.sync_copy(ei_hbm.at[pl.ds(ebase, EPW)], idxi_all)
    pltpu.sync_copy(ej_hbm.at[pl.ds(ebase, EPW)], idxj_all)
    lane = lax.iota(jnp.int32, 16)

    def start(c, ri, rj, si, sj):
        pltpu.async_copy(z_hbm.at[idxi_all.at[pl.ds(c * CHUNK, CHUNK)]], ri, si)
        pltpu.async_copy(z_hbm.at[idxj_all.at[pl.ds(c * CHUNK, CHUNK)]], rj, sj)

    def wait(c, ri, rj, si, sj):
        pltpu.make_async_copy(
            z_hbm.at[idxi_all.at[pl.ds(c * CHUNK, CHUNK)]], ri, si).wait()
        pltpu.make_async_copy(
            z_hbm.at[idxj_all.at[pl.ds(c * CHUNK, CHUNK)]], rj, sj).wait()

    def unpack2(v32):
        vbf = plsc.bitcast(v32, jnp.bfloat16)
        return plsc.unpack(vbf, format=plsc.PackFormat.INTERLEAVED)

    def compute(c, ri, rj):
        def group_body(g, carry):
            e_idx = lane + g * 16
            accs = [jnp.zeros((16,), jnp.float32) for _ in range(2)]
            for blk in range(DP // 16):
                for t in range(16):
                    dv = (lane ^ t) + blk * 16
                    via, vib = unpack2(plsc.load_gather(ri, [e_idx, dv]))
                    vja, vjb = unpack2(plsc.load_gather(rj, [e_idx, dv]))
                    accs[0] = accs[0] + via * vja
                    accs[1] = accs[1] + vib * vjb
            outv[pl.ds(c * CHUNK + g * 16, 16)] = accs[0] + accs[1]
            return carry

        lax.fori_loop(0, CHUNK // 16, group_body, 0)

    start(0, ri_a, rj_a, si_a, sj_a)

    def body2(t2, carry):
        t = 2 * t2
        start(t + 1, ri_b, rj_b, si_b, sj_b)
        wait(t, ri_a, rj_a, si_a, sj_a)
        compute(t, ri_a, rj_a)
        start(t + 2, ri_a, rj_a, si_a, sj_a)
        wait(t + 1, ri_b, rj_b, si_b, sj_b)
        compute(t + 1, ri_b, rj_b)
        return carry

    lax.fori_loop(0, (NCHUNK - 1) // 2, body2, 0)
    wait(NCHUNK - 1, ri_a, rj_a, si_a, sj_a)
    compute(NCHUNK - 1, ri_a, rj_a)
    pltpu.sync_copy(outv, out_hbm.at[pl.ds(ebase, EPW)])


@jax.jit
def kernel(z, edge_index):
    ei = edge_index[0].astype(jnp.int32)
    ej = edge_index[1].astype(jnp.int32)
    zp = lax.bitcast_convert_type(
        z.astype(jnp.bfloat16).reshape(z.shape[0], DP, 2), jnp.int32)
    mesh = plsc.VectorSubcoreMesh(core_axis_name="c", subcore_axis_name="s")
    f = functools.partial(
        pl.kernel,
        mesh=mesh,
        out_type=jax.ShapeDtypeStruct((E_TOTAL,), jnp.float32),
        scratch_types=[
            pltpu.VMEM((EPW,), jnp.int32),
            pltpu.VMEM((EPW,), jnp.int32),
            pltpu.VMEM((CHUNK, DP), jnp.int32),
            pltpu.VMEM((CHUNK, DP), jnp.int32),
            pltpu.VMEM((CHUNK, DP), jnp.int32),
            pltpu.VMEM((CHUNK, DP), jnp.int32),
            pltpu.VMEM((EPW,), jnp.float32),
            pltpu.SemaphoreType.DMA,
            pltpu.SemaphoreType.DMA,
            pltpu.SemaphoreType.DMA,
            pltpu.SemaphoreType.DMA,
        ],
        compiler_params=pltpu.CompilerParams(
            needs_layout_passes=False, use_tc_tiling_on_sc=False),
    )(_dot_body)
    return f(ei, ej, zp)
